# baseline (speedup 1.0000x reference)
import functools

import jax
import jax.numpy as jnp
from jax import lax
from jax.experimental import pallas as pl
from jax.experimental.pallas import tpu as pltpu
from jax.experimental.pallas import tpu_sc as plsc

B, N, S, D1, D2, DO = 8, 4096, 1024, 128, 256, 128
NBLK = 1024
NB = N // NBLK
BN = B * N
BS = B * S

_NC, _NS = 2, 16
NW = _NC * _NS
PPW = BN // NW
P = 128
NCHUNK = PPW // P


def _prep_body(f2_ref, x2t_ref, w1_ref, b1_ref, w2_ref, b2_ref,
               proj2_ref, w_ref, bias_ref, x2m2_ref, n2_ref):
    w = jnp.dot(w2_ref[...], w1_ref[...], preferred_element_type=jnp.float32)
    w_ref[...] = w
    bias_ref[...] = jnp.dot(w2_ref[...], b1_ref[...],
                            preferred_element_type=jnp.float32) + b2_ref[...]
    wb = w[:, D1:]
    proj2_ref[0] = jnp.dot(f2_ref[0], wb.T, preferred_element_type=jnp.float32)
    x2t = x2t_ref[0]
    n2_ref[0] = jnp.sum(x2t * x2t, axis=0, keepdims=True)
    x2m2_ref[0] = -2.0 * x2t


def _knn_body(x1_ref, x2m2_ref, n2_ref, i0_ref, i1_ref, i2_ref):
    b = pl.program_id(0)
    d0 = n2_ref[0] + jnp.dot(x1_ref[0], x2m2_ref[0],
                             preferred_element_type=jnp.float32)
    iota = jax.lax.broadcasted_iota(jnp.int32, (NBLK, S), 1)
    base = b * S
    d = d0
    for k, ref in enumerate((i0_ref, i1_ref, i2_ref)):
        m = jnp.min(d, axis=1, keepdims=True)
        r = jnp.min(jnp.where(d == m, iota, jnp.int32(S)), axis=1,
                    keepdims=True)
        ref[0] = r + base
        if k < 2:
            d = jnp.where(iota == r, jnp.float32(jnp.inf), d)


def _combine_body(f1_ref, g_ref, w_ref, bias_ref, out_ref):
    base_t = jax.lax.dot_general(
        w_ref[:, :D1], f1_ref[0], (((1,), (1,)), ((), ())),
        preferred_element_type=jnp.float32)
    out_ref[0] = base_t + g_ref[0].T * (1.0 / 3.0) + bias_ref[...]


def _gather_body(table_hbm, i0_hbm, i1_hbm, i2_hbm, out_hbm,
                 i0v, i1v, i2v, r0, r1, r2, ov, sem):
    wid = lax.axis_index("s") * _NC + lax.axis_index("c")
    wbase = wid * PPW

    def chunk(ci, carry):
        off = wbase + ci * P
        pltpu.sync_copy(i0_hbm.at[pl.ds(off, P)], i0v)
        pltpu.sync_copy(i1_hbm.at[pl.ds(off, P)], i1v)
        pltpu.sync_copy(i2_hbm.at[pl.ds(off, P)], i2v)
        c0 = pltpu.async_copy(table_hbm.at[i0v], r0, sem)
        c1 = pltpu.async_copy(table_hbm.at[i1v], r1, sem)
        c2 = pltpu.async_copy(table_hbm.at[i2v], r2, sem)
        c0.wait()
        c1.wait()
        c2.wait()

        def row(p, c):
            for j in range(DO // 16):
                sl = pl.ds(j * 16, 16)
                ov[p, sl] = r0[p, sl] + r1[p, sl] + r2[p, sl]
            return c

        lax.fori_loop(0, P, row, 0)
        pltpu.sync_copy(ov, out_hbm.at[pl.ds(off, P)])
        return carry

    lax.fori_loop(0, NCHUNK, chunk, 0)


def kernel(xyz1, xyz2, features1, features2, W1, b1, W2, b2):
    xyz1p = jnp.pad(xyz1, ((0, 0), (0, 0), (0, 5)))
    xyz2t = jnp.pad(xyz2, ((0, 0), (0, 0), (0, 5)))
    xyz2t = jnp.transpose(xyz2t, (0, 2, 1))
    b1r = b1.reshape(D2, 1)
    b2r = b2.reshape(DO, 1)

    proj2, w, bias, x2m2, n2 = pl.pallas_call(
        _prep_body,
        grid=(B,),
        in_specs=[
            pl.BlockSpec((1, S, D2), lambda b: (b, 0, 0)),
            pl.BlockSpec((1, 8, S), lambda b: (b, 0, 0)),
            pl.BlockSpec((D2, D1 + D2), lambda b: (0, 0)),
            pl.BlockSpec((D2, 1), lambda b: (0, 0)),
            pl.BlockSpec((DO, D2), lambda b: (0, 0)),
            pl.BlockSpec((DO, 1), lambda b: (0, 0)),
        ],
        out_specs=[
            pl.BlockSpec((1, S, DO), lambda b: (b, 0, 0)),
            pl.BlockSpec((DO, D1 + D2), lambda b: (0, 0)),
            pl.BlockSpec((DO, 1), lambda b: (0, 0)),
            pl.BlockSpec((1, 8, S), lambda b: (b, 0, 0)),
            pl.BlockSpec((1, 1, S), lambda b: (b, 0, 0)),
        ],
        out_shape=[
            jax.ShapeDtypeStruct((B, S, DO), jnp.float32),
            jax.ShapeDtypeStruct((DO, D1 + D2), jnp.float32),
            jax.ShapeDtypeStruct((DO, 1), jnp.float32),
            jax.ShapeDtypeStruct((B, 8, S), jnp.float32),
            jax.ShapeDtypeStruct((B, 1, S), jnp.float32),
        ],
    )(features2, xyz2t, W1, b1r, W2, b2r)

    idx_specs = pl.BlockSpec((1, NBLK, 1), lambda b, nb: (b * NB + nb, 0, 0))
    idx_shape = jax.ShapeDtypeStruct((B * NB, NBLK, 1), jnp.int32)
    i0, i1, i2 = pl.pallas_call(
        _knn_body,
        grid=(B, NB),
        in_specs=[
            pl.BlockSpec((1, NBLK, 8), lambda b, nb: (b, nb, 0)),
            pl.BlockSpec((1, 8, S), lambda b, nb: (b, 0, 0)),
            pl.BlockSpec((1, 1, S), lambda b, nb: (b, 0, 0)),
        ],
        out_specs=[idx_specs, idx_specs, idx_specs],
        out_shape=[idx_shape, idx_shape, idx_shape],
    )(xyz1p, x2m2, n2)

    table = proj2.reshape(BS, DO)
    i0f = i0.reshape(BN)
    i1f = i1.reshape(BN)
    i2f = i2.reshape(BN)

    mesh = plsc.VectorSubcoreMesh(core_axis_name="c", subcore_axis_name="s")
    gsum = functools.partial(
        pl.kernel,
        mesh=mesh,
        out_type=jax.ShapeDtypeStruct((BN, DO), jnp.float32),
        scratch_types=[
            pltpu.VMEM((P,), jnp.int32),
            pltpu.VMEM((P,), jnp.int32),
            pltpu.VMEM((P,), jnp.int32),
            pltpu.VMEM((P, DO), jnp.float32),
            pltpu.VMEM((P, DO), jnp.float32),
            pltpu.VMEM((P, DO), jnp.float32),
            pltpu.VMEM((P, DO), jnp.float32),
            pltpu.SemaphoreType.DMA,
        ],
    )(_gather_body)(table, i0f, i1f, i2f)

    gsum = gsum.reshape(B, N, DO)

    out = pl.pallas_call(
        _combine_body,
        grid=(B, NB),
        in_specs=[
            pl.BlockSpec((1, NBLK, D1), lambda b, nb: (b, nb, 0)),
            pl.BlockSpec((1, NBLK, DO), lambda b, nb: (b, nb, 0)),
            pl.BlockSpec((DO, D1 + D2), lambda b, nb: (0, 0)),
            pl.BlockSpec((DO, 1), lambda b, nb: (0, 0)),
        ],
        out_specs=pl.BlockSpec((1, DO, NBLK), lambda b, nb: (b, 0, nb)),
        out_shape=jax.ShapeDtypeStruct((B, DO, N), jnp.float32),
    )(features1, gsum, w, bias)
    return out


# speedup vs baseline: 1.0715x; 1.0715x over previous
import functools

import jax
import jax.numpy as jnp
from jax import lax
from jax.experimental import pallas as pl
from jax.experimental.pallas import tpu as pltpu
from jax.experimental.pallas import tpu_sc as plsc

B, N, S, D1, D2, DO = 8, 4096, 1024, 128, 256, 128
NBLK = 1024
NB = N // NBLK
BN = B * N
BS = B * S

_NC, _NS = 2, 16
NW = _NC * _NS
PPW = BN // NW
P = 128
NCHUNK = PPW // P


def _prep_body(f2_ref, x2t_ref, w1_ref, b1_ref, w2_ref, b2_ref,
               proj2_ref, w_ref, bias_ref, x2m2_ref, n2_ref):
    w = jnp.dot(w2_ref[...], w1_ref[...], preferred_element_type=jnp.float32)
    w_ref[...] = w
    bias_ref[...] = jnp.dot(w2_ref[...], b1_ref[...],
                            preferred_element_type=jnp.float32) + b2_ref[...]
    wb = w[:, D1:]
    proj2_ref[0] = jnp.dot(f2_ref[0], wb.T, preferred_element_type=jnp.float32)
    x2t = x2t_ref[0]
    n2_ref[0] = jnp.sum(x2t * x2t, axis=0, keepdims=True)
    x2m2_ref[0] = -2.0 * x2t


def _knn_body(x1_ref, x2m2_ref, n2_ref, i0_ref, i1_ref, i2_ref):
    b = pl.program_id(0)
    d0 = n2_ref[0] + jnp.dot(x1_ref[0], x2m2_ref[0],
                             preferred_element_type=jnp.float32)
    iota = jax.lax.broadcasted_iota(jnp.int32, (NBLK, S), 1)
    base = b * S
    d = d0
    for k, ref in enumerate((i0_ref, i1_ref, i2_ref)):
        m = jnp.min(d, axis=1, keepdims=True)
        r = jnp.min(jnp.where(d == m, iota, jnp.int32(S)), axis=1,
                    keepdims=True)
        ref[0] = r + base
        if k < 2:
            d = jnp.where(iota == r, jnp.float32(jnp.inf), d)


def _combine_body(f1_ref, g_ref, w_ref, bias_ref, out_ref):
    base_t = jax.lax.dot_general(
        w_ref[:, :D1], f1_ref[0], (((1,), (1,)), ((), ())),
        preferred_element_type=jnp.float32)
    out_ref[0] = base_t + g_ref[0].T * (1.0 / 3.0) + bias_ref[...]


def _gather_body(table_hbm, i0_hbm, i1_hbm, i2_hbm, out_hbm,
                 i0a, i1a, i2a, i0b, i1b, i2b,
                 r0a, r1a, r2a, r0b, r1b, r2b,
                 sga, sgb, ssa, ssb):
    wid = lax.axis_index("s") * _NC + lax.axis_index("c")
    wbase = wid * PPW
    idx = ((i0a, i1a, i2a), (i0b, i1b, i2b))
    rows = ((r0a, r1a, r2a), (r0b, r1b, r2b))
    gsem = (sga, sgb)
    ssem = (ssa, ssb)

    def fire(ci, s):
        off = wbase + ci * P
        for iv, ih in zip(idx[s], (i0_hbm, i1_hbm, i2_hbm)):
            pltpu.sync_copy(ih.at[pl.ds(off, P)], iv)
        return [pltpu.async_copy(table_hbm.at[iv], rv, gsem[s])
                for iv, rv in zip(idx[s], rows[s])]

    gh = {0: fire(0, 0)}
    sh = {}
    for ci in range(NCHUNK):
        s = ci % 2
        if ci + 1 < NCHUNK:
            sn = (ci + 1) % 2
            if ci - 1 >= 0:
                sh.pop(ci - 1).wait()
            gh[ci + 1] = fire(ci + 1, sn)
        for h in gh.pop(ci):
            h.wait()
        r0, r1, r2 = rows[s]

        def row(pp, c):
            for dp in range(2):
                p = pp * 2 + dp
                for j in range(DO // 16):
                    sl = pl.ds(j * 16, 16)
                    r0[p, sl] = r0[p, sl] + r1[p, sl] + r2[p, sl]
            return c

        lax.fori_loop(0, P // 2, row, 0)
        sh[ci] = pltpu.async_copy(
            r0, out_hbm.at[pl.ds(wbase + ci * P, P)], ssem[s])
    sh.pop(NCHUNK - 2).wait()
    sh.pop(NCHUNK - 1).wait()


def kernel(xyz1, xyz2, features1, features2, W1, b1, W2, b2):
    xyz1p = jnp.pad(xyz1, ((0, 0), (0, 0), (0, 5)))
    xyz2t = jnp.pad(xyz2, ((0, 0), (0, 0), (0, 5)))
    xyz2t = jnp.transpose(xyz2t, (0, 2, 1))
    b1r = b1.reshape(D2, 1)
    b2r = b2.reshape(DO, 1)

    proj2, w, bias, x2m2, n2 = pl.pallas_call(
        _prep_body,
        grid=(B,),
        in_specs=[
            pl.BlockSpec((1, S, D2), lambda b: (b, 0, 0)),
            pl.BlockSpec((1, 8, S), lambda b: (b, 0, 0)),
            pl.BlockSpec((D2, D1 + D2), lambda b: (0, 0)),
            pl.BlockSpec((D2, 1), lambda b: (0, 0)),
            pl.BlockSpec((DO, D2), lambda b: (0, 0)),
            pl.BlockSpec((DO, 1), lambda b: (0, 0)),
        ],
        out_specs=[
            pl.BlockSpec((1, S, DO), lambda b: (b, 0, 0)),
            pl.BlockSpec((DO, D1 + D2), lambda b: (0, 0)),
            pl.BlockSpec((DO, 1), lambda b: (0, 0)),
            pl.BlockSpec((1, 8, S), lambda b: (b, 0, 0)),
            pl.BlockSpec((1, 1, S), lambda b: (b, 0, 0)),
        ],
        out_shape=[
            jax.ShapeDtypeStruct((B, S, DO), jnp.float32),
            jax.ShapeDtypeStruct((DO, D1 + D2), jnp.float32),
            jax.ShapeDtypeStruct((DO, 1), jnp.float32),
            jax.ShapeDtypeStruct((B, 8, S), jnp.float32),
            jax.ShapeDtypeStruct((B, 1, S), jnp.float32),
        ],
    )(features2, xyz2t, W1, b1r, W2, b2r)

    idx_specs = pl.BlockSpec((1, NBLK, 1), lambda b, nb: (b * NB + nb, 0, 0))
    idx_shape = jax.ShapeDtypeStruct((B * NB, NBLK, 1), jnp.int32)
    i0, i1, i2 = pl.pallas_call(
        _knn_body,
        grid=(B, NB),
        in_specs=[
            pl.BlockSpec((1, NBLK, 8), lambda b, nb: (b, nb, 0)),
            pl.BlockSpec((1, 8, S), lambda b, nb: (b, 0, 0)),
            pl.BlockSpec((1, 1, S), lambda b, nb: (b, 0, 0)),
        ],
        out_specs=[idx_specs, idx_specs, idx_specs],
        out_shape=[idx_shape, idx_shape, idx_shape],
    )(xyz1p, x2m2, n2)

    table = proj2.reshape(BS, DO)
    i0f = i0.reshape(BN)
    i1f = i1.reshape(BN)
    i2f = i2.reshape(BN)

    mesh = plsc.VectorSubcoreMesh(core_axis_name="c", subcore_axis_name="s")
    gsum = functools.partial(
        pl.kernel,
        mesh=mesh,
        out_type=jax.ShapeDtypeStruct((BN, DO), jnp.float32),
        scratch_types=(
            [pltpu.VMEM((P,), jnp.int32)] * 6
            + [pltpu.VMEM((P, DO), jnp.float32)] * 6
            + [pltpu.SemaphoreType.DMA] * 4
        ),
    )(_gather_body)(table, i0f, i1f, i2f)

    gsum = gsum.reshape(B, N, DO)

    out = pl.pallas_call(
        _combine_body,
        grid=(B, NB),
        in_specs=[
            pl.BlockSpec((1, NBLK, D1), lambda b, nb: (b, nb, 0)),
            pl.BlockSpec((1, NBLK, DO), lambda b, nb: (b, nb, 0)),
            pl.BlockSpec((DO, D1 + D2), lambda b, nb: (0, 0)),
            pl.BlockSpec((DO, 1), lambda b, nb: (0, 0)),
        ],
        out_specs=pl.BlockSpec((1, DO, NBLK), lambda b, nb: (b, 0, nb)),
        out_shape=jax.ShapeDtypeStruct((B, DO, N), jnp.float32),
    )(features1, gsum, w, bias)
    return out


# speedup vs baseline: 1.0832x; 1.0109x over previous
import functools

import jax
import jax.numpy as jnp
from jax import lax
from jax.experimental import pallas as pl
from jax.experimental.pallas import tpu as pltpu
from jax.experimental.pallas import tpu_sc as plsc

B, N, S, D1, D2, DO = 8, 4096, 1024, 128, 256, 128
NBLK = 1024
NB = N // NBLK
BN = B * N
BS = B * S

_NC, _NS = 2, 16
NW = _NC * _NS
PPW = BN // NW
P = 128
NCHUNK = PPW // P


def _prep_body(f2_ref, x2t_ref, w1_ref, b1_ref, w2_ref, b2_ref,
               proj2_ref, w_ref, bias_ref, x2m2_ref, n2_ref):
    w = jnp.dot(w2_ref[...], w1_ref[...], preferred_element_type=jnp.float32)
    w_ref[...] = w
    bias_ref[...] = jnp.dot(w2_ref[...], b1_ref[...],
                            preferred_element_type=jnp.float32) + b2_ref[...]
    wb = w[:, D1:]
    proj2_ref[0] = jnp.dot(f2_ref[0], wb.T, preferred_element_type=jnp.float32)
    x2t = x2t_ref[0]
    n2_ref[0] = jnp.sum(x2t * x2t, axis=0, keepdims=True)
    x2m2_ref[0] = -2.0 * x2t


def _knn_body(x1_ref, x2m2_ref, n2_ref, i0_ref, i1_ref, i2_ref):
    b = pl.program_id(0)
    d0 = n2_ref[0] + jnp.dot(x1_ref[0], x2m2_ref[0],
                             preferred_element_type=jnp.float32)
    iota = jax.lax.broadcasted_iota(jnp.int32, (NBLK, S), 1)
    base = b * S
    d = d0
    for k, ref in enumerate((i0_ref, i1_ref, i2_ref)):
        m = jnp.min(d, axis=1, keepdims=True)
        sel = d == m
        r = jnp.min(jnp.where(sel, iota, jnp.int32(S)), axis=1,
                    keepdims=True)
        ref[0] = r + base
        if k < 2:
            d = jnp.where(sel, jnp.float32(jnp.inf), d)


def _combine_body(f1_ref, g_ref, w_ref, bias_ref, out_ref):
    base_t = jax.lax.dot_general(
        w_ref[:, :D1], f1_ref[0], (((1,), (1,)), ((), ())),
        preferred_element_type=jnp.float32)
    out_ref[0] = base_t + g_ref[0].T * (1.0 / 3.0) + bias_ref[...]


def _gather_body(table_hbm, i0_hbm, i1_hbm, i2_hbm, out_hbm,
                 i0a, i1a, i2a, i0b, i1b, i2b,
                 r0a, r1a, r2a, r0b, r1b, r2b,
                 sga, sgb, ssa, ssb):
    wid = lax.axis_index("s") * _NC + lax.axis_index("c")
    wbase = wid * PPW
    idx = ((i0a, i1a, i2a), (i0b, i1b, i2b))
    rows = ((r0a, r1a, r2a), (r0b, r1b, r2b))
    gsem = (sga, sgb)
    ssem = (ssa, ssb)

    def fire(ci, s):
        off = wbase + ci * P
        for iv, ih in zip(idx[s], (i0_hbm, i1_hbm, i2_hbm)):
            pltpu.sync_copy(ih.at[pl.ds(off, P)], iv)
        return [pltpu.async_copy(table_hbm.at[iv], rv, gsem[s])
                for iv, rv in zip(idx[s], rows[s])]

    gh = {0: fire(0, 0)}
    sh = {}
    for ci in range(NCHUNK):
        s = ci % 2
        if ci + 1 < NCHUNK:
            sn = (ci + 1) % 2
            if ci - 1 >= 0:
                sh.pop(ci - 1).wait()
            gh[ci + 1] = fire(ci + 1, sn)
        for h in gh.pop(ci):
            h.wait()
        r0, r1, r2 = rows[s]

        def row(pp, c):
            for dp in range(2):
                p = pp * 2 + dp
                for j in range(DO // 16):
                    sl = pl.ds(j * 16, 16)
                    r0[p, sl] = r0[p, sl] + r1[p, sl] + r2[p, sl]
            return c

        lax.fori_loop(0, P // 2, row, 0)
        sh[ci] = pltpu.async_copy(
            r0, out_hbm.at[pl.ds(wbase + ci * P, P)], ssem[s])
    sh.pop(NCHUNK - 2).wait()
    sh.pop(NCHUNK - 1).wait()


def kernel(xyz1, xyz2, features1, features2, W1, b1, W2, b2):
    xyz1p = jnp.pad(xyz1, ((0, 0), (0, 0), (0, 5)))
    xyz2t = jnp.pad(xyz2, ((0, 0), (0, 0), (0, 5)))
    xyz2t = jnp.transpose(xyz2t, (0, 2, 1))
    b1r = b1.reshape(D2, 1)
    b2r = b2.reshape(DO, 1)

    proj2, w, bias, x2m2, n2 = pl.pallas_call(
        _prep_body,
        grid=(B,),
        in_specs=[
            pl.BlockSpec((1, S, D2), lambda b: (b, 0, 0)),
            pl.BlockSpec((1, 8, S), lambda b: (b, 0, 0)),
            pl.BlockSpec((D2, D1 + D2), lambda b: (0, 0)),
            pl.BlockSpec((D2, 1), lambda b: (0, 0)),
            pl.BlockSpec((DO, D2), lambda b: (0, 0)),
            pl.BlockSpec((DO, 1), lambda b: (0, 0)),
        ],
        out_specs=[
            pl.BlockSpec((1, S, DO), lambda b: (b, 0, 0)),
            pl.BlockSpec((DO, D1 + D2), lambda b: (0, 0)),
            pl.BlockSpec((DO, 1), lambda b: (0, 0)),
            pl.BlockSpec((1, 8, S), lambda b: (b, 0, 0)),
            pl.BlockSpec((1, 1, S), lambda b: (b, 0, 0)),
        ],
        out_shape=[
            jax.ShapeDtypeStruct((B, S, DO), jnp.float32),
            jax.ShapeDtypeStruct((DO, D1 + D2), jnp.float32),
            jax.ShapeDtypeStruct((DO, 1), jnp.float32),
            jax.ShapeDtypeStruct((B, 8, S), jnp.float32),
            jax.ShapeDtypeStruct((B, 1, S), jnp.float32),
        ],
    )(features2, xyz2t, W1, b1r, W2, b2r)

    idx_specs = pl.BlockSpec((1, NBLK, 1), lambda b, nb: (b * NB + nb, 0, 0))
    idx_shape = jax.ShapeDtypeStruct((B * NB, NBLK, 1), jnp.int32)
    i0, i1, i2 = pl.pallas_call(
        _knn_body,
        grid=(B, NB),
        in_specs=[
            pl.BlockSpec((1, NBLK, 8), lambda b, nb: (b, nb, 0)),
            pl.BlockSpec((1, 8, S), lambda b, nb: (b, 0, 0)),
            pl.BlockSpec((1, 1, S), lambda b, nb: (b, 0, 0)),
        ],
        out_specs=[idx_specs, idx_specs, idx_specs],
        out_shape=[idx_shape, idx_shape, idx_shape],
    )(xyz1p, x2m2, n2)

    table = proj2.reshape(BS, DO)
    i0f = i0.reshape(BN)
    i1f = i1.reshape(BN)
    i2f = i2.reshape(BN)

    mesh = plsc.VectorSubcoreMesh(core_axis_name="c", subcore_axis_name="s")
    gsum = functools.partial(
        pl.kernel,
        mesh=mesh,
        out_type=jax.ShapeDtypeStruct((BN, DO), jnp.float32),
        scratch_types=(
            [pltpu.VMEM((P,), jnp.int32)] * 6
            + [pltpu.VMEM((P, DO), jnp.float32)] * 6
            + [pltpu.SemaphoreType.DMA] * 4
        ),
    )(_gather_body)(table, i0f, i1f, i2f)

    gsum = gsum.reshape(B, N, DO)

    out = pl.pallas_call(
        _combine_body,
        grid=(B, NB),
        in_specs=[
            pl.BlockSpec((1, NBLK, D1), lambda b, nb: (b, nb, 0)),
            pl.BlockSpec((1, NBLK, DO), lambda b, nb: (b, nb, 0)),
            pl.BlockSpec((DO, D1 + D2), lambda b, nb: (0, 0)),
            pl.BlockSpec((DO, 1), lambda b, nb: (0, 0)),
        ],
        out_specs=pl.BlockSpec((1, DO, NBLK), lambda b, nb: (b, 0, nb)),
        out_shape=jax.ShapeDtypeStruct((B, DO, N), jnp.float32),
    )(features1, gsum, w, bias)
    return out
